# Initial kernel scaffold; baseline (speedup 1.0000x reference)
import jax
import jax.numpy as jnp
from jax.experimental import pallas as pl


def kernel(cl_idx, cc_feat, al_idx, ac_feat, test_idx, es0, es1, es2, enc_cl_w, enc_cc_W, enc_cc_b, enc_al_w, enc_ac_W, enc_ac_b, emb_test_w, mp_W_tgt, mp_b_tgt, mp_W_src, mp_b_src, mp_emb_se, dec_W, dec_b):
    raise NotImplementedError("write your pallas kernel here")


# speedup vs baseline: 1.2716x; 1.2716x over previous
import functools
import jax
import jax.numpy as jnp
from jax import lax
from jax.experimental import pallas as pl
from jax.experimental.pallas import tpu as pltpu
import jax.experimental.pallas.tpu_sc as plsc

N = 10000
E = 160000
D = 256
NCL = 1000
NW = 32
RPW = 320
TRASH = RPW
PE = 161088
NEG = -3.4e38

def _mesh():
    return plsc.VectorSubcoreMesh(core_axis_name="c", subcore_axis_name="s",
                                  num_cores=2, num_subcores=16)


def _wid():
    return lax.axis_index("s") * 2 + lax.axis_index("c")


@functools.cache
def _enc_gather_kernel():
    return pl.kernel(
        _enc_gather_body,
        out_type=(jax.ShapeDtypeStruct((320 * NW, D), jnp.float32),
                  jax.ShapeDtypeStruct((320 * NW, D), jnp.float32)),
        mesh=_mesh(),
        scratch_types=[pltpu.VMEM((64,), jnp.int32),
                       pltpu.VMEM((64, D), jnp.float32),
                       pltpu.SemaphoreType.DMA],
        compiler_params=pltpu.CompilerParams(use_tc_tiling_on_sc=False,
                                             needs_layout_passes=False),
    )


def _enc_gather_body(clw, cli, alw, ali, ocl, oal, ib, rb, sem):
    w = _wid()
    for tab, idx, out in ((clw, cli, ocl), (alw, ali, oal)):
        for t in range(5):
            base = pl.multiple_of(w * 320 + t * 64, 64)
            pltpu.sync_copy(idx.at[pl.ds(base, 64)], ib)
            pltpu.async_copy(tab.at[ib], rb, sem).wait()
            pltpu.sync_copy(rb, out.at[pl.ds(base, 64)])


@functools.cache
def _seg_max_kernel():
    return pl.kernel(
        _seg_max_body,
        out_type=jax.ShapeDtypeStruct((NW * RPW * D,), jnp.float32),
        mesh=_mesh(),
        scratch_types=[pltpu.VMEM(((RPW + 1) * D,), jnp.float32),
                       pltpu.VMEM((16,), jnp.int32),
                       pltpu.VMEM((16,), jnp.int32),
                       pltpu.VMEM((16,), jnp.int32),
                       pltpu.VMEM((16,), jnp.int32),
                       pltpu.VMEM((16, D), jnp.float32),
                       pltpu.VMEM((16, D), jnp.float32),
                       pltpu.VMEM((16,), jnp.int32),
                       pltpu.SemaphoreType.DMA,
                       pltpu.SemaphoreType.DMA],
        compiler_params=pltpu.CompilerParams(use_tc_tiling_on_sc=False,
                                             needs_layout_passes=False),
    )


def _seg_max_body(tt, eg, eo, meta, init, out,
                  acc, gb0, gb1, ob0, ob1, r0, r1, mb, sem0, sem1):
    w = _wid()
    iota = lax.iota(jnp.int32, 16)
    bufs = ((gb0, ob0, r0, sem0), (gb1, ob1, r1, sem1))

    pltpu.sync_copy(meta.at[w], mb)
    mv = mb[...]
    nb = mv[0]
    eb = pl.multiple_of(mv[1], 32)
    pltpu.sync_copy(init, acc)

    for b, (gb, ob, rr, sem) in enumerate(bufs):
        pb = pl.multiple_of(eb + b * 16, 16)
        pltpu.sync_copy(eg.at[pl.ds(pb, 16)], gb)
        pltpu.sync_copy(eo.at[pl.ds(pb, 16)], ob)
        pltpu.async_copy(tt.at[gb], rr, sem)

    @pl.loop(0, nb, step=2)
    def _(t):
        for b, (gb, ob, rr, sem) in enumerate(bufs):
            blk = t + b
            pltpu.make_async_copy(tt.at[gb], rr, sem).wait()
            ov = ob[...]
            for e in range(16):
                rowv = ov.at[jnp.full((16,), e, jnp.int32)].get(
                    mode="promise_in_bounds") * D
                for c in range(16):
                    idxv = rowv + (c * 16 + iota)
                    a = plsc.load_gather(acc, [idxv])
                    rv = rr[e, pl.ds(c * 16, 16)]
                    plsc.store_scatter(acc, [idxv], jnp.maximum(a, rv))
            nxt = pl.multiple_of(eb + (blk + 2) * 16, 16)
            pltpu.sync_copy(eg.at[pl.ds(nxt, 16)], gb)
            pltpu.sync_copy(eo.at[pl.ds(nxt, 16)], ob)
            pltpu.async_copy(tt.at[gb], rr, sem)

    for b, (gb, ob, rr, sem) in enumerate(bufs):
        pltpu.make_async_copy(tt.at[gb], rr, sem).wait()
    pltpu.sync_copy(acc.at[pl.ds(0, RPW * D)],
                    out.at[pl.ds(w * (RPW * D), RPW * D)])


def _enc_body(cc, ac, gcl, gal, wcc, wac, bcc, bac, emt, x_ref):
    dn = (((1,), (1,)), ((), ()))
    x_ref[0] = gcl[...] + lax.dot_general(cc[...], wcc[...], dn,
                                          preferred_element_type=jnp.float32) + bcc[...]
    x_ref[1] = gal[...] + lax.dot_general(ac[...], wac[...], dn,
                                          preferred_element_type=jnp.float32) + bac[...]
    x_ref[2] = jnp.broadcast_to(emt[...], (1000, D))


def _tc_encode(cc, ac, gcl, gal, wcc, wac, bcc, bac, emt):
    blk = lambda *shape: pl.BlockSpec(shape, lambda b: (0,) * len(shape))
    return pl.pallas_call(
        _enc_body,
        grid=(10,),
        in_specs=[
            pl.BlockSpec((1000, 128), lambda b: (b, 0)),
            pl.BlockSpec((1000, 128), lambda b: (b, 0)),
            pl.BlockSpec((1000, D), lambda b: (b, 0)),
            pl.BlockSpec((1000, D), lambda b: (b, 0)),
            blk(D, 128), blk(D, 128), blk(1, D), blk(1, D), blk(1, D),
        ],
        out_specs=pl.BlockSpec((3, 1000, D), lambda b: (0, b, 0)),
        out_shape=jax.ShapeDtypeStruct((3, N, D), jnp.float32),
    )(cc, ac, gcl, gal, wcc, wac, bcc, bac, emt)


T_SRC = (0, 1, 2)
T_TGT = (1, 2, 0)


def _tables_body(x, wt, bt, ws, bs, se, y_ref, tt0, tt1, tt2):
    dn = (((1,), (1,)), ((), ()))
    trefs = (tt0, tt1, tt2)
    for j in range(3):
        xt = x[T_TGT[j]]
        xs = x[T_SRC[j]]
        y_ref[j] = lax.dot_general(xt, wt[j], dn,
                                   preferred_element_type=jnp.float32) + bt[j]
        z = lax.dot_general(xs, ws[j], dn,
                            preferred_element_type=jnp.float32) + bs[j]
        trefs[j][0] = z + se[j, 0]
        trefs[j][1] = z + se[j, 1]


def _tc_tables(x, wt, bt, ws, bs, se):
    blk = lambda *shape: pl.BlockSpec(shape, lambda b: (0,) * len(shape))
    tshape = jax.ShapeDtypeStruct((2, N, D), jnp.float32)
    return pl.pallas_call(
        _tables_body,
        grid=(10,),
        in_specs=[
            pl.BlockSpec((3, 1000, D), lambda b: (0, b, 0)),
            blk(3, D, D), blk(3, 1, D), blk(3, D, D), blk(3, 1, D),
            blk(3, 2, D),
        ],
        out_specs=[
            pl.BlockSpec((3, 1000, D), lambda b: (0, b, 0)),
            pl.BlockSpec((2, 1000, D), lambda b: (0, b, 0)),
            pl.BlockSpec((2, 1000, D), lambda b: (0, b, 0)),
            pl.BlockSpec((2, 1000, D), lambda b: (0, b, 0)),
        ],
        out_shape=[jax.ShapeDtypeStruct((3, N, D), jnp.float32),
                   tshape, tshape, tshape],
    )(x, wt, bt, ws, bs, se)


def _combine_body(y, s0, s1, s2, x_ref):
    x_ref[0] = jnp.maximum(y[2] + s2[...], 0.0)
    x_ref[1] = jnp.maximum(y[0] + s0[...], 0.0)
    x_ref[2] = jnp.maximum(y[1] + s1[...], 0.0)


def _tc_combine(y, s0, s1, s2):
    seg_spec = pl.BlockSpec((1000, D), lambda b: (b, 0))
    return pl.pallas_call(
        _combine_body,
        grid=(10,),
        in_specs=[pl.BlockSpec((3, 1000, D), lambda b: (0, b, 0)),
                  seg_spec, seg_spec, seg_spec],
        out_specs=pl.BlockSpec((3, 1000, D), lambda b: (0, b, 0)),
        out_shape=jax.ShapeDtypeStruct((3, N, D), jnp.float32),
    )(y, s0, s1, s2)


def _dec_body(x, wp, bp, last_ref, probs_ref):
    dn = (((1,), (1,)), ((), ()))
    l = lax.dot_general(x[0], wp[...], dn,
                        preferred_element_type=jnp.float32) + bp[...]
    mask = lax.broadcasted_iota(jnp.int32, (1000, 128), 1) < 3
    m = jnp.max(jnp.where(mask, l, NEG), axis=1, keepdims=True)
    e = jnp.where(mask, jnp.exp(l - m), 0.0)
    probs_ref[...] = e / jnp.sum(e, axis=1, keepdims=True)
    last_ref[...] = l


def _tc_decode(x, wp, bp):
    blk = lambda *shape: pl.BlockSpec(shape, lambda b: (0,) * len(shape))
    oshape = jax.ShapeDtypeStruct((3 * N, 128), jnp.float32)
    return pl.pallas_call(
        _dec_body,
        grid=(30,),
        in_specs=[pl.BlockSpec((1, 1000, D), lambda b: (b // 10, b % 10, 0)),
                  blk(128, D), blk(1, 128)],
        out_specs=[pl.BlockSpec((1000, 128), lambda b: (b, 0)),
                   pl.BlockSpec((1000, 128), lambda b: (b, 0))],
        out_shape=[oshape, oshape],
    )(x, wp, bp)


def _edge_meta(es):
    src = es[0].astype(jnp.int32)
    dst = es[1].astype(jnp.int32)
    g = src + jnp.where(src == dst, N, 0).astype(jnp.int32)
    dst_s, g_s = lax.sort((dst, g), num_keys=1)
    bounds = jnp.arange(NW + 1, dtype=jnp.int32) * RPW
    edges = jnp.searchsorted(dst_s, bounds).astype(jnp.int32)
    starts = edges[:-1]
    counts = edges[1:] - starts
    cpad = jnp.maximum(((counts + 31) // 32) * 32, 32)
    offs = jnp.concatenate([jnp.zeros((1,), jnp.int32),
                            jnp.cumsum(cpad)[:-1].astype(jnp.int32)])
    i = jnp.arange(PE, dtype=jnp.int32)
    w_i = jnp.searchsorted(offs, i, side="right").astype(jnp.int32) - 1
    r = i - offs[w_i]
    valid = r < counts[w_i]
    k = jnp.clip(starts[w_i] + r, 0, E - 1)
    eg = jnp.where(valid, g_s[k], 0)
    rel = dst_s[k] - w_i * RPW
    eo = jnp.where(valid, rel, TRASH)
    meta = jnp.zeros((NW, 16), jnp.int32)
    meta = meta.at[:, 0].set(cpad // 16).at[:, 1].set(offs)
    return eg, eo, meta


def kernel(cl_idx, cc_feat, al_idx, ac_feat, test_idx, es0, es1, es2,
           enc_cl_w, enc_cc_W, enc_cc_b, enc_al_w, enc_ac_W, enc_ac_b,
           emb_test_w, mp_W_tgt, mp_b_tgt, mp_W_src, mp_b_src, mp_emb_se,
           dec_W, dec_b):
    f32 = jnp.float32
    eg0, eo0, m0 = _edge_meta(es0)
    eg1, eo1, m1 = _edge_meta(es1)
    eg2, eo2, m2 = _edge_meta(es2)
    init = jnp.full(((RPW + 1) * D,), NEG, f32)

    pad_idx = lambda ix: jnp.pad(ix.astype(jnp.int32), (0, 320 * NW - N))
    gcl, gal = _enc_gather_kernel()(enc_cl_w.astype(f32), pad_idx(cl_idx),
                                    enc_al_w.astype(f32), pad_idx(al_idx))

    x = _tc_encode(cc_feat, ac_feat, gcl[:N], gal[:N],
                   enc_cc_W, enc_ac_W,
                   enc_cc_b.reshape(1, D), enc_ac_b.reshape(1, D),
                   emb_test_w.reshape(1, D))

    for i in range(5):
        y, t0, t1, t2 = _tc_tables(
            x, mp_W_tgt[i], mp_b_tgt[i].reshape(3, 1, D),
            mp_W_src[i], mp_b_src[i].reshape(3, 1, D), mp_emb_se[i])
        sk = _seg_max_kernel()
        s0 = sk(t0.reshape(2 * N, D), eg0, eo0, m0, init)
        s1 = sk(t1.reshape(2 * N, D), eg1, eo1, m1, init)
        s2 = sk(t2.reshape(2 * N, D), eg2, eo2, m2, init)
        x = _tc_combine(y, s0.reshape(NW * RPW, D)[:N],
                        s1.reshape(NW * RPW, D)[:N],
                        s2.reshape(NW * RPW, D)[:N])

    wp = jnp.zeros((128, D), f32).at[:3].set(dec_W)
    bp = jnp.zeros((1, 128), f32).at[0, :3].set(dec_b)
    last_p, probs_p = _tc_decode(x, wp, bp)
    return last_p[:, :3], probs_p[:, :3]


# speedup vs baseline: 1.6421x; 1.2913x over previous
import functools
import jax
import jax.numpy as jnp
from jax import lax
from jax.experimental import pallas as pl
from jax.experimental.pallas import tpu as pltpu
import jax.experimental.pallas.tpu_sc as plsc

N = 10000
E = 160000
D = 256
NCL = 1000
NW = 32
RPW = 320
TRASH = RPW
PE = 161088
NEG = -3.4e38

def _mesh():
    return plsc.VectorSubcoreMesh(core_axis_name="c", subcore_axis_name="s",
                                  num_cores=2, num_subcores=16)


def _wid():
    return lax.axis_index("s") * 2 + lax.axis_index("c")


@functools.cache
def _enc_gather_kernel():
    return pl.kernel(
        _enc_gather_body,
        out_type=(jax.ShapeDtypeStruct((320 * NW, D), jnp.float32),
                  jax.ShapeDtypeStruct((320 * NW, D), jnp.float32)),
        mesh=_mesh(),
        scratch_types=[pltpu.VMEM((64,), jnp.int32),
                       pltpu.VMEM((64, D), jnp.float32),
                       pltpu.SemaphoreType.DMA],
        compiler_params=pltpu.CompilerParams(use_tc_tiling_on_sc=False,
                                             needs_layout_passes=False),
    )


def _enc_gather_body(clw, cli, alw, ali, ocl, oal, ib, rb, sem):
    w = _wid()
    for tab, idx, out in ((clw, cli, ocl), (alw, ali, oal)):
        for t in range(5):
            base = pl.multiple_of(w * 320 + t * 64, 64)
            pltpu.sync_copy(idx.at[pl.ds(base, 64)], ib)
            pltpu.async_copy(tab.at[ib], rb, sem).wait()
            pltpu.sync_copy(rb, out.at[pl.ds(base, 64)])


@functools.cache
def _seg_max_kernel():
    return pl.kernel(
        _seg_max_body,
        out_type=jax.ShapeDtypeStruct((NW * RPW * D,), jnp.float32),
        mesh=_mesh(),
        scratch_types=[pltpu.VMEM(((RPW + 1) * D,), jnp.float32),
                       pltpu.VMEM((16,), jnp.int32),
                       pltpu.VMEM((16,), jnp.int32),
                       pltpu.VMEM((16,), jnp.int32),
                       pltpu.VMEM((16,), jnp.int32),
                       pltpu.VMEM((16, D), jnp.float32),
                       pltpu.VMEM((16, D), jnp.float32),
                       pltpu.VMEM((16,), jnp.int32),
                       pltpu.SemaphoreType.DMA,
                       pltpu.SemaphoreType.DMA],
        compiler_params=pltpu.CompilerParams(use_tc_tiling_on_sc=False,
                                             needs_layout_passes=False),
    )


def _seg_max_body(tt, eg, eo, meta, init, out,
                  acc, gb0, gb1, ob0, ob1, r0, r1, mb, sem0, sem1):
    w = _wid()
    bufs = ((gb0, ob0, r0, sem0), (gb1, ob1, r1, sem1))

    pltpu.sync_copy(meta.at[w], mb)
    mv = mb[...]
    nb = mv[0]
    eb = pl.multiple_of(mv[1], 32)
    pltpu.sync_copy(init, acc)

    for b, (gb, ob, rr, sem) in enumerate(bufs):
        pb = pl.multiple_of(eb + b * 16, 16)
        pltpu.sync_copy(eg.at[pl.ds(pb, 16)], gb)
        pltpu.sync_copy(eo.at[pl.ds(pb, 16)], ob)
        pltpu.async_copy(tt.at[gb], rr, sem)

    @pl.loop(0, nb, step=2)
    def _(t):
        for b, (gb, ob, rr, sem) in enumerate(bufs):
            blk = t + b
            pltpu.make_async_copy(tt.at[gb], rr, sem).wait()
            ov = ob[...]
            for e in range(16):
                base = ov[e] * D
                for c in range(16):
                    off = pl.multiple_of(base + c * 16, 16)
                    a = acc[pl.ds(off, 16)]
                    rv = rr[e, pl.ds(c * 16, 16)]
                    acc[pl.ds(off, 16)] = jnp.maximum(a, rv)
            nxt = pl.multiple_of(eb + (blk + 2) * 16, 16)
            pltpu.sync_copy(eg.at[pl.ds(nxt, 16)], gb)
            pltpu.sync_copy(eo.at[pl.ds(nxt, 16)], ob)
            pltpu.async_copy(tt.at[gb], rr, sem)

    for b, (gb, ob, rr, sem) in enumerate(bufs):
        pltpu.make_async_copy(tt.at[gb], rr, sem).wait()
    pltpu.sync_copy(acc.at[pl.ds(0, RPW * D)],
                    out.at[pl.ds(w * (RPW * D), RPW * D)])


def _enc_body(cc, ac, gcl, gal, wcc, wac, bcc, bac, emt, x_ref):
    dn = (((1,), (1,)), ((), ()))
    x_ref[0] = gcl[...] + lax.dot_general(cc[...], wcc[...], dn,
                                          preferred_element_type=jnp.float32) + bcc[...]
    x_ref[1] = gal[...] + lax.dot_general(ac[...], wac[...], dn,
                                          preferred_element_type=jnp.float32) + bac[...]
    x_ref[2] = jnp.broadcast_to(emt[...], (1000, D))


def _tc_encode(cc, ac, gcl, gal, wcc, wac, bcc, bac, emt):
    blk = lambda *shape: pl.BlockSpec(shape, lambda b: (0,) * len(shape))
    return pl.pallas_call(
        _enc_body,
        grid=(10,),
        in_specs=[
            pl.BlockSpec((1000, 128), lambda b: (b, 0)),
            pl.BlockSpec((1000, 128), lambda b: (b, 0)),
            pl.BlockSpec((1000, D), lambda b: (b, 0)),
            pl.BlockSpec((1000, D), lambda b: (b, 0)),
            blk(D, 128), blk(D, 128), blk(1, D), blk(1, D), blk(1, D),
        ],
        out_specs=pl.BlockSpec((3, 1000, D), lambda b: (0, b, 0)),
        out_shape=jax.ShapeDtypeStruct((3, N, D), jnp.float32),
    )(cc, ac, gcl, gal, wcc, wac, bcc, bac, emt)


T_SRC = (0, 1, 2)
T_TGT = (1, 2, 0)


def _tables_body(x, wt, bt, ws, bs, se, y_ref, tt0, tt1, tt2):
    dn = (((1,), (1,)), ((), ()))
    trefs = (tt0, tt1, tt2)
    for j in range(3):
        xt = x[T_TGT[j]]
        xs = x[T_SRC[j]]
        y_ref[j] = lax.dot_general(xt, wt[j], dn,
                                   preferred_element_type=jnp.float32) + bt[j]
        z = lax.dot_general(xs, ws[j], dn,
                            preferred_element_type=jnp.float32) + bs[j]
        trefs[j][0] = z + se[j, 0]
        trefs[j][1] = z + se[j, 1]


def _tc_tables(x, wt, bt, ws, bs, se):
    blk = lambda *shape: pl.BlockSpec(shape, lambda b: (0,) * len(shape))
    tshape = jax.ShapeDtypeStruct((2, N, D), jnp.float32)
    return pl.pallas_call(
        _tables_body,
        grid=(10,),
        in_specs=[
            pl.BlockSpec((3, 1000, D), lambda b: (0, b, 0)),
            blk(3, D, D), blk(3, 1, D), blk(3, D, D), blk(3, 1, D),
            blk(3, 2, D),
        ],
        out_specs=[
            pl.BlockSpec((3, 1000, D), lambda b: (0, b, 0)),
            pl.BlockSpec((2, 1000, D), lambda b: (0, b, 0)),
            pl.BlockSpec((2, 1000, D), lambda b: (0, b, 0)),
            pl.BlockSpec((2, 1000, D), lambda b: (0, b, 0)),
        ],
        out_shape=[jax.ShapeDtypeStruct((3, N, D), jnp.float32),
                   tshape, tshape, tshape],
    )(x, wt, bt, ws, bs, se)


def _combine_body(y, s0, s1, s2, x_ref):
    x_ref[0] = jnp.maximum(y[2] + s2[...], 0.0)
    x_ref[1] = jnp.maximum(y[0] + s0[...], 0.0)
    x_ref[2] = jnp.maximum(y[1] + s1[...], 0.0)


def _tc_combine(y, s0, s1, s2):
    seg_spec = pl.BlockSpec((1000, D), lambda b: (b, 0))
    return pl.pallas_call(
        _combine_body,
        grid=(10,),
        in_specs=[pl.BlockSpec((3, 1000, D), lambda b: (0, b, 0)),
                  seg_spec, seg_spec, seg_spec],
        out_specs=pl.BlockSpec((3, 1000, D), lambda b: (0, b, 0)),
        out_shape=jax.ShapeDtypeStruct((3, N, D), jnp.float32),
    )(y, s0, s1, s2)


def _dec_body(x, wp, bp, last_ref, probs_ref):
    dn = (((1,), (1,)), ((), ()))
    l = lax.dot_general(x[0], wp[...], dn,
                        preferred_element_type=jnp.float32) + bp[...]
    mask = lax.broadcasted_iota(jnp.int32, (1000, 128), 1) < 3
    m = jnp.max(jnp.where(mask, l, NEG), axis=1, keepdims=True)
    e = jnp.where(mask, jnp.exp(l - m), 0.0)
    probs_ref[...] = e / jnp.sum(e, axis=1, keepdims=True)
    last_ref[...] = l


def _tc_decode(x, wp, bp):
    blk = lambda *shape: pl.BlockSpec(shape, lambda b: (0,) * len(shape))
    oshape = jax.ShapeDtypeStruct((3 * N, 128), jnp.float32)
    return pl.pallas_call(
        _dec_body,
        grid=(30,),
        in_specs=[pl.BlockSpec((1, 1000, D), lambda b: (b // 10, b % 10, 0)),
                  blk(128, D), blk(1, 128)],
        out_specs=[pl.BlockSpec((1000, 128), lambda b: (b, 0)),
                   pl.BlockSpec((1000, 128), lambda b: (b, 0))],
        out_shape=[oshape, oshape],
    )(x, wp, bp)


def _edge_meta(es):
    src = es[0].astype(jnp.int32)
    dst = es[1].astype(jnp.int32)
    g = src + jnp.where(src == dst, N, 0).astype(jnp.int32)
    dst_s, g_s = lax.sort((dst, g), num_keys=1)
    bounds = jnp.arange(NW + 1, dtype=jnp.int32) * RPW
    edges = jnp.searchsorted(dst_s, bounds).astype(jnp.int32)
    starts = edges[:-1]
    counts = edges[1:] - starts
    cpad = jnp.maximum(((counts + 31) // 32) * 32, 32)
    offs = jnp.concatenate([jnp.zeros((1,), jnp.int32),
                            jnp.cumsum(cpad)[:-1].astype(jnp.int32)])
    i = jnp.arange(PE, dtype=jnp.int32)
    w_i = jnp.searchsorted(offs, i, side="right").astype(jnp.int32) - 1
    r = i - offs[w_i]
    valid = r < counts[w_i]
    k = jnp.clip(starts[w_i] + r, 0, E - 1)
    eg = jnp.where(valid, g_s[k], 0)
    rel = dst_s[k] - w_i * RPW
    eo = jnp.where(valid, rel, TRASH)
    meta = jnp.zeros((NW, 16), jnp.int32)
    meta = meta.at[:, 0].set(cpad // 16).at[:, 1].set(offs)
    return eg, eo, meta


def kernel(cl_idx, cc_feat, al_idx, ac_feat, test_idx, es0, es1, es2,
           enc_cl_w, enc_cc_W, enc_cc_b, enc_al_w, enc_ac_W, enc_ac_b,
           emb_test_w, mp_W_tgt, mp_b_tgt, mp_W_src, mp_b_src, mp_emb_se,
           dec_W, dec_b):
    f32 = jnp.float32
    eg0, eo0, m0 = _edge_meta(es0)
    eg1, eo1, m1 = _edge_meta(es1)
    eg2, eo2, m2 = _edge_meta(es2)
    init = jnp.full(((RPW + 1) * D,), NEG, f32)

    pad_idx = lambda ix: jnp.pad(ix.astype(jnp.int32), (0, 320 * NW - N))
    gcl, gal = _enc_gather_kernel()(enc_cl_w.astype(f32), pad_idx(cl_idx),
                                    enc_al_w.astype(f32), pad_idx(al_idx))

    x = _tc_encode(cc_feat, ac_feat, gcl[:N], gal[:N],
                   enc_cc_W, enc_ac_W,
                   enc_cc_b.reshape(1, D), enc_ac_b.reshape(1, D),
                   emb_test_w.reshape(1, D))

    for i in range(5):
        y, t0, t1, t2 = _tc_tables(
            x, mp_W_tgt[i], mp_b_tgt[i].reshape(3, 1, D),
            mp_W_src[i], mp_b_src[i].reshape(3, 1, D), mp_emb_se[i])
        sk = _seg_max_kernel()
        s0 = sk(t0.reshape(2 * N, D), eg0, eo0, m0, init)
        s1 = sk(t1.reshape(2 * N, D), eg1, eo1, m1, init)
        s2 = sk(t2.reshape(2 * N, D), eg2, eo2, m2, init)
        x = _tc_combine(y, s0.reshape(NW * RPW, D)[:N],
                        s1.reshape(NW * RPW, D)[:N],
                        s2.reshape(NW * RPW, D)[:N])

    wp = jnp.zeros((128, D), f32).at[:3].set(dec_W)
    bp = jnp.zeros((1, 128), f32).at[0, :3].set(dec_b)
    last_p, probs_p = _tc_decode(x, wp, bp)
    return last_p[:, :3], probs_p[:, :3]


# speedup vs baseline: 1.7875x; 1.0886x over previous
import functools
import jax
import jax.numpy as jnp
from jax import lax
from jax.experimental import pallas as pl
from jax.experimental.pallas import tpu as pltpu
import jax.experimental.pallas.tpu_sc as plsc

N = 10000
E = 160000
D = 256
NCL = 1000
NW = 32
RPW = 320
TRASH = RPW
PE = 162176
NEG = -3.4e38

def _mesh():
    return plsc.VectorSubcoreMesh(core_axis_name="c", subcore_axis_name="s",
                                  num_cores=2, num_subcores=16)


def _wid():
    return lax.axis_index("s") * 2 + lax.axis_index("c")


@functools.cache
def _enc_gather_kernel():
    return pl.kernel(
        _enc_gather_body,
        out_type=(jax.ShapeDtypeStruct((320 * NW, D), jnp.float32),
                  jax.ShapeDtypeStruct((320 * NW, D), jnp.float32)),
        mesh=_mesh(),
        scratch_types=[pltpu.VMEM((64,), jnp.int32),
                       pltpu.VMEM((64, D), jnp.float32),
                       pltpu.SemaphoreType.DMA],
        compiler_params=pltpu.CompilerParams(use_tc_tiling_on_sc=False,
                                             needs_layout_passes=False),
    )


def _enc_gather_body(clw, cli, alw, ali, ocl, oal, ib, rb, sem):
    w = _wid()
    for tab, idx, out in ((clw, cli, ocl), (alw, ali, oal)):
        for t in range(5):
            base = pl.multiple_of(w * 320 + t * 64, 64)
            pltpu.sync_copy(idx.at[pl.ds(base, 64)], ib)
            pltpu.async_copy(tab.at[ib], rb, sem).wait()
            pltpu.sync_copy(rb, out.at[pl.ds(base, 64)])


@functools.cache
def _seg_max_kernel():
    return pl.kernel(
        _seg_max_body,
        out_type=jax.ShapeDtypeStruct((NW * RPW * D,), jnp.float32),
        mesh=_mesh(),
        scratch_types=[pltpu.VMEM(((RPW + 1) * D,), jnp.float32),
                       pltpu.VMEM((32,), jnp.int32),
                       pltpu.VMEM((32,), jnp.int32),
                       pltpu.VMEM((32,), jnp.int32),
                       pltpu.VMEM((32,), jnp.int32),
                       pltpu.VMEM((32, D), jnp.float32),
                       pltpu.VMEM((32, D), jnp.float32),
                       pltpu.VMEM((16,), jnp.int32),
                       pltpu.SemaphoreType.DMA,
                       pltpu.SemaphoreType.DMA],
        compiler_params=pltpu.CompilerParams(use_tc_tiling_on_sc=False,
                                             needs_layout_passes=False),
    )


def _seg_max_body(tt, eg, eo, meta, init, out,
                  acc, gb0, gb1, ob0, ob1, r0, r1, mb, sem0, sem1):
    w = _wid()
    bufs = ((gb0, ob0, r0, sem0), (gb1, ob1, r1, sem1))

    pltpu.sync_copy(meta.at[w], mb)
    mv = mb[...]
    nb = mv[0]
    eb = pl.multiple_of(mv[1], 64)
    pltpu.sync_copy(init, acc)

    for b, (gb, ob, rr, sem) in enumerate(bufs):
        pb = pl.multiple_of(eb + b * 32, 32)
        pltpu.sync_copy(eg.at[pl.ds(pb, 32)], gb)
        pltpu.sync_copy(eo.at[pl.ds(pb, 32)], ob)
        pltpu.async_copy(tt.at[gb], rr, sem)

    @pl.loop(0, nb, step=2)
    def _(t):
        for b, (gb, ob, rr, sem) in enumerate(bufs):
            blk = t + b
            pltpu.make_async_copy(tt.at[gb], rr, sem).wait()
            ov0 = ob[pl.ds(0, 16)]
            ov1 = ob[pl.ds(16, 16)]
            for e in range(32):
                base = (ov0 if e < 16 else ov1)[e % 16] * D
                for c in range(16):
                    off = pl.multiple_of(base + c * 16, 16)
                    a = acc[pl.ds(off, 16)]
                    rv = rr[e, pl.ds(c * 16, 16)]
                    acc[pl.ds(off, 16)] = jnp.maximum(a, rv)
            nxt = pl.multiple_of(eb + (blk + 2) * 32, 32)
            pltpu.sync_copy(eg.at[pl.ds(nxt, 32)], gb)
            pltpu.sync_copy(eo.at[pl.ds(nxt, 32)], ob)
            pltpu.async_copy(tt.at[gb], rr, sem)

    for b, (gb, ob, rr, sem) in enumerate(bufs):
        pltpu.make_async_copy(tt.at[gb], rr, sem).wait()
    pltpu.sync_copy(acc.at[pl.ds(0, RPW * D)],
                    out.at[pl.ds(w * (RPW * D), RPW * D)])


def _enc_body(cc, ac, gcl, gal, wcc, wac, bcc, bac, emt, x_ref):
    dn = (((1,), (1,)), ((), ()))
    x_ref[0] = gcl[...] + lax.dot_general(cc[...], wcc[...], dn,
                                          preferred_element_type=jnp.float32) + bcc[...]
    x_ref[1] = gal[...] + lax.dot_general(ac[...], wac[...], dn,
                                          preferred_element_type=jnp.float32) + bac[...]
    x_ref[2] = jnp.broadcast_to(emt[...], (1000, D))


def _tc_encode(cc, ac, gcl, gal, wcc, wac, bcc, bac, emt):
    blk = lambda *shape: pl.BlockSpec(shape, lambda b: (0,) * len(shape))
    return pl.pallas_call(
        _enc_body,
        grid=(10,),
        in_specs=[
            pl.BlockSpec((1000, 128), lambda b: (b, 0)),
            pl.BlockSpec((1000, 128), lambda b: (b, 0)),
            pl.BlockSpec((1000, D), lambda b: (b, 0)),
            pl.BlockSpec((1000, D), lambda b: (b, 0)),
            blk(D, 128), blk(D, 128), blk(1, D), blk(1, D), blk(1, D),
        ],
        out_specs=pl.BlockSpec((3, 1000, D), lambda b: (0, b, 0)),
        out_shape=jax.ShapeDtypeStruct((3, N, D), jnp.float32),
    )(cc, ac, gcl, gal, wcc, wac, bcc, bac, emt)


T_SRC = (0, 1, 2)
T_TGT = (1, 2, 0)


def _tables_body(x, wt, bt, ws, bs, se, y_ref, tt0, tt1, tt2):
    dn = (((1,), (1,)), ((), ()))
    trefs = (tt0, tt1, tt2)
    for j in range(3):
        xt = x[T_TGT[j]]
        xs = x[T_SRC[j]]
        y_ref[j] = lax.dot_general(xt, wt[j], dn,
                                   preferred_element_type=jnp.float32) + bt[j]
        z = lax.dot_general(xs, ws[j], dn,
                            preferred_element_type=jnp.float32) + bs[j]
        trefs[j][0] = z + se[j, 0]
        trefs[j][1] = z + se[j, 1]


def _tc_tables(x, wt, bt, ws, bs, se):
    blk = lambda *shape: pl.BlockSpec(shape, lambda b: (0,) * len(shape))
    tshape = jax.ShapeDtypeStruct((2, N, D), jnp.float32)
    return pl.pallas_call(
        _tables_body,
        grid=(10,),
        in_specs=[
            pl.BlockSpec((3, 1000, D), lambda b: (0, b, 0)),
            blk(3, D, D), blk(3, 1, D), blk(3, D, D), blk(3, 1, D),
            blk(3, 2, D),
        ],
        out_specs=[
            pl.BlockSpec((3, 1000, D), lambda b: (0, b, 0)),
            pl.BlockSpec((2, 1000, D), lambda b: (0, b, 0)),
            pl.BlockSpec((2, 1000, D), lambda b: (0, b, 0)),
            pl.BlockSpec((2, 1000, D), lambda b: (0, b, 0)),
        ],
        out_shape=[jax.ShapeDtypeStruct((3, N, D), jnp.float32),
                   tshape, tshape, tshape],
    )(x, wt, bt, ws, bs, se)


def _combine_body(y, s0, s1, s2, x_ref):
    x_ref[0] = jnp.maximum(y[2] + s2[...], 0.0)
    x_ref[1] = jnp.maximum(y[0] + s0[...], 0.0)
    x_ref[2] = jnp.maximum(y[1] + s1[...], 0.0)


def _tc_combine(y, s0, s1, s2):
    seg_spec = pl.BlockSpec((1000, D), lambda b: (b, 0))
    return pl.pallas_call(
        _combine_body,
        grid=(10,),
        in_specs=[pl.BlockSpec((3, 1000, D), lambda b: (0, b, 0)),
                  seg_spec, seg_spec, seg_spec],
        out_specs=pl.BlockSpec((3, 1000, D), lambda b: (0, b, 0)),
        out_shape=jax.ShapeDtypeStruct((3, N, D), jnp.float32),
    )(y, s0, s1, s2)


def _dec_body(x, wp, bp, last_ref, probs_ref):
    dn = (((1,), (1,)), ((), ()))
    l = lax.dot_general(x[0], wp[...], dn,
                        preferred_element_type=jnp.float32) + bp[...]
    mask = lax.broadcasted_iota(jnp.int32, (1000, 128), 1) < 3
    m = jnp.max(jnp.where(mask, l, NEG), axis=1, keepdims=True)
    e = jnp.where(mask, jnp.exp(l - m), 0.0)
    probs_ref[...] = e / jnp.sum(e, axis=1, keepdims=True)
    last_ref[...] = l


def _tc_decode(x, wp, bp):
    blk = lambda *shape: pl.BlockSpec(shape, lambda b: (0,) * len(shape))
    oshape = jax.ShapeDtypeStruct((3 * N, 128), jnp.float32)
    return pl.pallas_call(
        _dec_body,
        grid=(30,),
        in_specs=[pl.BlockSpec((1, 1000, D), lambda b: (b // 10, b % 10, 0)),
                  blk(128, D), blk(1, 128)],
        out_specs=[pl.BlockSpec((1000, 128), lambda b: (b, 0)),
                   pl.BlockSpec((1000, 128), lambda b: (b, 0))],
        out_shape=[oshape, oshape],
    )(x, wp, bp)


def _edge_meta(es):
    src = es[0].astype(jnp.int32)
    dst = es[1].astype(jnp.int32)
    g = src + jnp.where(src == dst, N, 0).astype(jnp.int32)
    dst_s, g_s = lax.sort((dst, g), num_keys=1)
    bounds = jnp.arange(NW + 1, dtype=jnp.int32) * RPW
    edges = jnp.searchsorted(dst_s, bounds).astype(jnp.int32)
    starts = edges[:-1]
    counts = edges[1:] - starts
    cpad = jnp.maximum(((counts + 63) // 64) * 64, 64)
    offs = jnp.concatenate([jnp.zeros((1,), jnp.int32),
                            jnp.cumsum(cpad)[:-1].astype(jnp.int32)])
    i = jnp.arange(PE, dtype=jnp.int32)
    w_i = jnp.searchsorted(offs, i, side="right").astype(jnp.int32) - 1
    r = i - offs[w_i]
    valid = r < counts[w_i]
    k = jnp.clip(starts[w_i] + r, 0, E - 1)
    eg = jnp.where(valid, g_s[k], 0)
    rel = dst_s[k] - w_i * RPW
    eo = jnp.where(valid, rel, TRASH)
    meta = jnp.zeros((NW, 16), jnp.int32)
    meta = meta.at[:, 0].set(cpad // 32).at[:, 1].set(offs)
    return eg, eo, meta


def kernel(cl_idx, cc_feat, al_idx, ac_feat, test_idx, es0, es1, es2,
           enc_cl_w, enc_cc_W, enc_cc_b, enc_al_w, enc_ac_W, enc_ac_b,
           emb_test_w, mp_W_tgt, mp_b_tgt, mp_W_src, mp_b_src, mp_emb_se,
           dec_W, dec_b):
    f32 = jnp.float32
    eg0, eo0, m0 = _edge_meta(es0)
    eg1, eo1, m1 = _edge_meta(es1)
    eg2, eo2, m2 = _edge_meta(es2)
    init = jnp.full(((RPW + 1) * D,), NEG, f32)

    pad_idx = lambda ix: jnp.pad(ix.astype(jnp.int32), (0, 320 * NW - N))
    gcl, gal = _enc_gather_kernel()(enc_cl_w.astype(f32), pad_idx(cl_idx),
                                    enc_al_w.astype(f32), pad_idx(al_idx))

    x = _tc_encode(cc_feat, ac_feat, gcl[:N], gal[:N],
                   enc_cc_W, enc_ac_W,
                   enc_cc_b.reshape(1, D), enc_ac_b.reshape(1, D),
                   emb_test_w.reshape(1, D))

    for i in range(5):
        y, t0, t1, t2 = _tc_tables(
            x, mp_W_tgt[i], mp_b_tgt[i].reshape(3, 1, D),
            mp_W_src[i], mp_b_src[i].reshape(3, 1, D), mp_emb_se[i])
        sk = _seg_max_kernel()
        s0 = sk(t0.reshape(2 * N, D), eg0, eo0, m0, init)
        s1 = sk(t1.reshape(2 * N, D), eg1, eo1, m1, init)
        s2 = sk(t2.reshape(2 * N, D), eg2, eo2, m2, init)
        x = _tc_combine(y, s0.reshape(NW * RPW, D)[:N],
                        s1.reshape(NW * RPW, D)[:N],
                        s2.reshape(NW * RPW, D)[:N])

    wp = jnp.zeros((128, D), f32).at[:3].set(dec_W)
    bp = jnp.zeros((1, 128), f32).at[0, :3].set(dec_b)
    last_p, probs_p = _tc_decode(x, wp, bp)
    return last_p[:, :3], probs_p[:, :3]
